# Initial kernel scaffold; baseline (speedup 1.0000x reference)
import jax
import jax.numpy as jnp
from jax.experimental import pallas as pl


def kernel(z, pos, edge_index, embedding, freqs, W_rbf, W_v, W_upd, W_vs):
    raise NotImplementedError("write your pallas kernel here")


# speedup vs baseline: 22.1390x; 22.1390x over previous
import functools

import jax
import jax.numpy as jnp
from jax import lax
from jax.experimental import pallas as pl
from jax.experimental.pallas import tpu as pltpu
from jax.experimental.pallas import tpu_sc as plsc

N = 10000
E = 320000
D = 128
DV = 32
NR = 12
CUT = 6.0
NTYPES = 100

NC = 2
NS = 16
NW = NC * NS
EW = E // NW
GCH = 400
GNC = EW // GCH
ECORE = E // NS
SCH = 80
SNC = ECORE // SCH
NPAD = 10240
ROWS_T = NPAD // NS

_mesh = plsc.VectorSubcoreMesh(core_axis_name="c", subcore_axis_name="s")
_sc_params = pltpu.CompilerParams(needs_layout_passes=False)


@functools.partial(
    pl.kernel,
    mesh=_mesh,
    out_type=jax.ShapeDtypeStruct((E * 8,), jnp.float32),
    scratch_types=[
        pltpu.VMEM((N * 3,), jnp.float32),
        pltpu.VMEM((N,), jnp.int32),
        pltpu.VMEM((GCH,), jnp.int32),
        pltpu.VMEM((GCH,), jnp.int32),
        pltpu.VMEM((GCH * 8,), jnp.float32),
    ],
    compiler_params=_sc_params,
)
def _edge_geom(src_hbm, dst_hbm, pos_hbm, z_hbm, out_hbm,
               pos_v, z_v, src_b, dst_b, stage):
    wid = lax.axis_index("s") * NC + lax.axis_index("c")
    pltpu.sync_copy(pos_hbm, pos_v)
    pltpu.sync_copy(z_hbm, z_v)
    lanes16 = lax.iota(jnp.int32, 16)

    def chunk_body(g, _):
        base = wid * EW + g * GCH
        pltpu.sync_copy(src_hbm.at[pl.ds(base, GCH)], src_b)
        pltpu.sync_copy(dst_hbm.at[pl.ds(base, GCH)], dst_b)

        def vec_body(v, _):
            off = v * 16
            srcv = src_b[pl.ds(off, 16)]
            dstv = dst_b[pl.ds(off, 16)]
            src3 = srcv * 3
            dst3 = dstv * 3
            col = (off + lanes16) * 8
            for k in range(3):
                ps = plsc.load_gather(pos_v, [src3 + k])
                pd = plsc.load_gather(pos_v, [dst3 + k])
                plsc.store_scatter(stage, [col + k], pd - ps)
            zi = plsc.load_gather(z_v, [srcv])
            plsc.store_scatter(stage, [col + 3], zi.astype(jnp.float32))
            return 0

        lax.fori_loop(0, GCH // 16, vec_body, 0)
        pltpu.sync_copy(stage, out_hbm.at[pl.ds(base * 8, GCH * 8)])
        return 0

    lax.fori_loop(0, GNC, chunk_body, 0)


BE = 1280


def _edge_dense_body(vecd_ref, emb_ref, wrbf_ref, wv_ref, freqs_ref,
                     msg_ref, vmsg_ref):
    blk = vecd_ref[...]
    vec = blk[:, 0:3]
    d2 = jnp.sum(vec * vec, axis=1, keepdims=True) + 1e-9
    dist = jnp.sqrt(d2)
    ds_ = dist * (1.0 / CUT)
    inv = 1.0 / ds_
    ds2 = ds_ * ds_
    ds4 = ds2 * ds2
    ds5 = ds4 * ds_
    env = inv + ds5 * (-28.0 + ds_ * (48.0 + ds_ * (-21.0)))
    env = jnp.where(ds_ < 1.0, env, 0.0)
    rbf = env * jnp.sin(ds_ * freqs_ref[...])
    gate = jnp.dot(rbf, wrbf_ref[...], preferred_element_type=jnp.float32)
    gate = gate * jax.nn.sigmoid(gate)
    zcol = blk[:, 3:4].astype(jnp.int32)
    onehot = (zcol == lax.broadcasted_iota(jnp.int32, (BE, D), 1))
    xs = jnp.dot(onehot.astype(jnp.float32), emb_ref[...],
                 preferred_element_type=jnp.float32)
    msg = xs * gate
    msg_ref[...] = msg
    mv = jnp.dot(msg, wv_ref[...], preferred_element_type=jnp.float32)
    unit = vec / dist
    parts = [unit[:, k:k + 1] * mv for k in range(3)]
    parts.append(jnp.zeros((BE, D - 3 * DV), jnp.float32))
    vmsg_ref[...] = jnp.concatenate(parts, axis=1)


def _edge_dense(vecd, embP, W_rbf, W_v, freqs_row):
    return pl.pallas_call(
        _edge_dense_body,
        grid=(E // BE,),
        in_specs=[
            pl.BlockSpec((BE, 8), lambda i: (i, 0)),
            pl.BlockSpec((D, D), lambda i: (0, 0)),
            pl.BlockSpec((NR, D), lambda i: (0, 0)),
            pl.BlockSpec((D, DV), lambda i: (0, 0)),
            pl.BlockSpec((1, NR), lambda i: (0, 0)),
        ],
        out_specs=[
            pl.BlockSpec((BE, D), lambda i: (i, 0)),
            pl.BlockSpec((BE, D), lambda i: (i, 0)),
        ],
        out_shape=[
            jax.ShapeDtypeStruct((E, D), jnp.float32),
            jax.ShapeDtypeStruct((E, D), jnp.float32),
        ],
    )(vecd, embP, W_rbf, W_v, freqs_row)


@functools.partial(
    pl.kernel,
    mesh=_mesh,
    out_type=[
        jax.ShapeDtypeStruct((NPAD, D), jnp.float32),
        jax.ShapeDtypeStruct((NPAD, D), jnp.float32),
    ],
    scratch_types=[
        pltpu.VMEM_SHARED((NPAD, D), jnp.float32),
        pltpu.VMEM((SCH,), jnp.int32),
        pltpu.VMEM((SCH, D), jnp.float32),
    ],
    compiler_params=_sc_params,
)
def _scatter(dst_hbm, msg_hbm, vmsg_hbm, zrows_hbm, agg_hbm, vagg_hbm,
             acc_sh, idx_b, row_b):
    cid = lax.axis_index("c")
    sid = lax.axis_index("s")
    pltpu.sync_copy(zrows_hbm, acc_sh.at[pl.ds(sid * ROWS_T, ROWS_T)])
    plsc.subcore_barrier()

    def make_loop(src_hbm):
        def body(g, _):
            base = sid * ECORE + g * SCH
            pltpu.sync_copy(dst_hbm.at[pl.ds(base, SCH)], idx_b)
            pltpu.sync_copy(src_hbm.at[pl.ds(base, SCH)], row_b)
            pltpu.sync_copy(row_b, acc_sh.at[idx_b], add=True)
            return 0
        return body

    @pl.when(cid == 0)
    def _():
        lax.fori_loop(0, SNC, make_loop(msg_hbm), 0)

    @pl.when(cid == 1)
    def _():
        lax.fori_loop(0, SNC, make_loop(vmsg_hbm), 0)

    plsc.subcore_barrier()
    rows = pl.ds(sid * ROWS_T, ROWS_T)

    @pl.when(cid == 0)
    def _():
        pltpu.sync_copy(acc_sh.at[rows], agg_hbm.at[rows])

    @pl.when(cid == 1)
    def _():
        pltpu.sync_copy(acc_sh.at[rows], vagg_hbm.at[rows])


BN = 2000


def _node_update_body(zf_ref, agg_ref, vagg_ref, emb_ref, wupd_ref, wvs_ref,
                      out_ref):
    onehot = (zf_ref[...].astype(jnp.int32)
              == lax.broadcasted_iota(jnp.int32, (BN, D), 1))
    xs = jnp.dot(onehot.astype(jnp.float32), emb_ref[...],
                 preferred_element_type=jnp.float32)
    h = jnp.dot(agg_ref[...], wupd_ref[...], preferred_element_type=jnp.float32)
    x_new = xs + h * jax.nn.sigmoid(h)
    va = vagg_ref[...]
    v0 = va[:, 0:DV]
    v1 = va[:, DV:2 * DV]
    v2 = va[:, 2 * DV:3 * DV]
    vnorm = jnp.sqrt(v0 * v0 + v1 * v1 + v2 * v2 + 1e-9)
    out_ref[...] = x_new + jnp.dot(vnorm, wvs_ref[...],
                                   preferred_element_type=jnp.float32)


def _node_update(zf, agg, vagg, embP, W_upd, W_vs):
    return pl.pallas_call(
        _node_update_body,
        grid=(N // BN,),
        in_specs=[
            pl.BlockSpec((BN, 1), lambda i: (i, 0)),
            pl.BlockSpec((BN, D), lambda i: (i, 0)),
            pl.BlockSpec((BN, D), lambda i: (i, 0)),
            pl.BlockSpec((D, D), lambda i: (0, 0)),
            pl.BlockSpec((D, D), lambda i: (0, 0)),
            pl.BlockSpec((DV, D), lambda i: (0, 0)),
        ],
        out_specs=pl.BlockSpec((BN, D), lambda i: (i, 0)),
        out_shape=jax.ShapeDtypeStruct((N, D), jnp.float32),
    )(zf, agg, vagg, embP, W_upd, W_vs)


def kernel(z, pos, edge_index, embedding, freqs, W_rbf, W_v, W_upd, W_vs):
    z = z.astype(jnp.int32)
    edge_index = edge_index.astype(jnp.int32)
    src = edge_index[0]
    dst = edge_index[1]
    pos = pos.astype(jnp.float32)

    vecd = _edge_geom(src, dst, pos.reshape(-1), z).reshape(E, 8)

    embP = jnp.zeros((D, D), jnp.float32).at[:NTYPES, :].set(embedding)
    msg, vmsg = _edge_dense(vecd, embP, W_rbf, W_v, freqs.reshape(1, NR))

    zrows = jnp.zeros((ROWS_T, D), jnp.float32)
    agg, vagg = _scatter(dst, msg, vmsg, zrows)
    agg = agg[:N]
    vagg = vagg[:N]

    zf = z.astype(jnp.float32).reshape(N, 1)
    out = _node_update(zf, agg, vagg, embP, W_upd, W_vs)
    return out
